# baseline (speedup 1.0000x reference)
import functools

import jax
import jax.numpy as jnp
from jax import lax
from jax.experimental import pallas as pl
from jax.experimental.pallas import tpu as pltpu
from jax.experimental.pallas import tpu_sc as plsc

N = 10000
E = 320000
D = 128
KHOPS = 4
DH = D // 2

NC = 2
NS = 16
NW = NC * NS
CW = 128
NBUF = 2
F0 = 0.75
C0 = int(round(E * F0 / (NS * CW * NBUF))) * NBUF
E0 = NS * C0 * CW
C1 = (-(-(E - E0) // (NS * CW * NBUF))) * NBUF
NCHUNKM = max(C0, C1)
SLAB = 632
SLABL = N - (NS - 1) * SLAB
NT = N + 8

BN = 1000
EPS = 1e-12

_SC_PARAMS = pltpu.CompilerParams(use_tc_tiling_on_sc=False)


def _per_tile_slab(s, fn):
    off = pl.multiple_of(s * SLAB, 8)

    @pl.when(s < NS - 1)
    def _():
        fn(off, SLAB)

    @pl.when(s == NS - 1)
    def _():
        fn(off, SLABL)


def _sc_mesh():
    return plsc.VectorSubcoreMesh(
        core_axis_name="c", subcore_axis_name="s",
        num_cores=NC, num_subcores=NS)


@functools.cache
def _build_deg_sc():
    return pl.kernel(
        _deg_sc_body,
        out_type=jax.ShapeDtypeStruct((NC, N, DH), jnp.float32),
        mesh=_sc_mesh(),
        compiler_params=_SC_PARAMS,
        scratch_types=[
            pltpu.VMEM((NCHUNKM, CW), jnp.int32),
            pltpu.VMEM((CW, DH), jnp.float32),
            pltpu.VMEM_SHARED((NT, DH), jnp.float32),
        ],
    )


def _deg_sc_body(rowr, zerosd, onesd, out, idx_v, ones_v, deg_sh):
    c = lax.axis_index("c")
    s = lax.axis_index("s")
    t = c * NS + s
    pltpu.sync_copy(rowr.at[t], idx_v)
    pltpu.sync_copy(onesd, ones_v)
    _per_tile_slab(s, lambda off, sz: pltpu.sync_copy(
        zerosd.at[pl.ds(off, sz)], deg_sh.at[pl.ds(off, sz)]))
    plsc.subcore_barrier()

    def body(j, carry):
        pltpu.sync_copy(ones_v, deg_sh.at[idx_v.at[j]], add=True)
        return carry

    lax.fori_loop(0, lax.select(c == 0, C0, C1), body, 0)
    plsc.subcore_barrier()
    _per_tile_slab(s, lambda off, sz: pltpu.sync_copy(
        deg_sh.at[pl.ds(off, sz)], out.at[c, pl.ds(off, sz)]))
    return


@functools.cache
def _build_spmv_sc():
    return pl.kernel(
        _spmv_sc_body,
        out_type=jax.ShapeDtypeStruct((NC, 2, N, DH), jnp.float32),
        mesh=_sc_mesh(),
        compiler_params=_SC_PARAMS,
        scratch_types=[
            pltpu.VMEM((NCHUNKM, CW), jnp.int32),
            pltpu.VMEM((NCHUNKM, CW), jnp.int32),
            [pltpu.VMEM((CW, DH), jnp.float32)] * NBUF,
            pltpu.VMEM_SHARED((NT, DH), jnp.float32),
            [pltpu.SemaphoreType.DMA] * NBUF,
        ],
    )


def _spmv_sc_body(g0, g1, srcr, dstr, zerosd, out,
                  src_v, dst_v, bufs, s_sh, sems):
    c = lax.axis_index("c")
    s = lax.axis_index("s")
    t = c * NS + s
    pltpu.sync_copy(srcr.at[t], src_v)
    pltpu.sync_copy(dstr.at[t], dst_v)
    knt = lax.select(c == 0, C0 // NBUF, C1 // NBUF)
    for p, g in enumerate((g0, g1)):
        _per_tile_slab(s, lambda off, sz: pltpu.sync_copy(
            zerosd.at[pl.ds(off, sz)], s_sh.at[pl.ds(off, sz)]))
        plsc.subcore_barrier()
        for b in range(NBUF - 1):
            pltpu.async_copy(g.at[src_v.at[b]], bufs[b], sems[b])

        def body(k, carry):
            for b in range(NBUF):
                j = NBUF * k + b
                nb = (b + NBUF - 1) % NBUF
                pltpu.make_async_copy(g.at[src_v.at[j]], bufs[b],
                                      sems[b]).wait()
                if b == 0:
                    pltpu.async_copy(g.at[src_v.at[j + NBUF - 1]],
                                     bufs[nb], sems[nb])
                else:
                    @pl.when(k < knt - 1)
                    def _():
                        pltpu.async_copy(g.at[src_v.at[j + NBUF - 1]],
                                         bufs[nb], sems[nb])
                pltpu.sync_copy(bufs[b], s_sh.at[dst_v.at[j]], add=True)
            return carry

        lax.fori_loop(0, knt, body, 0)
        plsc.subcore_barrier()
        _per_tile_slab(s, lambda off, sz: pltpu.sync_copy(
            s_sh.at[pl.ds(off, sz)], out.at[c, p, pl.ds(off, sz)]))
        plsc.subcore_barrier()
    return


def _deg_sc(*args):
    return _build_deg_sc()(*args)


def _spmv_sc(*args):
    return _build_spmv_sc()(*args)


def _rownorm_body(x_ref, dinv_ref, h_ref, g0_ref, g1_ref):
    xb = x_ref[...]
    ss = jnp.sum(xb * xb, axis=1, keepdims=True)
    inv = 1.0 / jnp.maximum(jnp.sqrt(ss), EPS)
    hb = xb * inv
    h_ref[...] = hb
    gb = hb * dinv_ref[...]
    g0_ref[...] = gb[:, :DH]
    g1_ref[...] = gb[:, DH:]


def _update_body(h_ref, dinv_ref, sp_ref, hn_ref, g0_ref, g1_ref):
    hb = h_ref[...]
    dv = dinv_ref[...]
    s = sp_ref[0] + sp_ref[1]
    sfull = jnp.concatenate([s[0], s[1]], axis=1)
    hn = 2.0 * hb - dv * sfull
    hn_ref[...] = hn
    gb = hn * dv
    g0_ref[...] = gb[:, :DH]
    g1_ref[...] = gb[:, DH:]


def _colss_body(lx_ref, ss_ref):
    i = pl.program_id(1)

    @pl.when(i == 0)
    def _():
        ss_ref[...] = jnp.zeros_like(ss_ref)

    xb = lx_ref[...]
    ss_ref[...] += jnp.sum(xb * xb, axis=1, keepdims=True)


def _scale_body(lx_ref, scl_ref, o_ref):
    o_ref[...] = lx_ref[...] * scl_ref[...]


_NB = N // BN

_rownorm = pl.pallas_call(
    _rownorm_body,
    grid=(_NB,),
    in_specs=[
        pl.BlockSpec((BN, D), lambda i: (i, 0)),
        pl.BlockSpec((BN, 1), lambda i: (i, 0)),
    ],
    out_specs=[
        pl.BlockSpec((BN, D), lambda i: (i, 0)),
        pl.BlockSpec((BN, DH), lambda i: (i, 0)),
        pl.BlockSpec((BN, DH), lambda i: (i, 0)),
    ],
    out_shape=[
        jax.ShapeDtypeStruct((N, D), jnp.float32),
        jax.ShapeDtypeStruct((N, DH), jnp.float32),
        jax.ShapeDtypeStruct((N, DH), jnp.float32),
    ],
)

_update = pl.pallas_call(
    _update_body,
    grid=(_NB,),
    in_specs=[
        pl.BlockSpec((BN, D), lambda i: (i, 0)),
        pl.BlockSpec((BN, 1), lambda i: (i, 0)),
        pl.BlockSpec((NC, 2, BN, DH), lambda i: (0, 0, i, 0)),
    ],
    out_specs=[
        pl.BlockSpec((BN, D), lambda i: (i, 0)),
        pl.BlockSpec((BN, DH), lambda i: (i, 0)),
        pl.BlockSpec((BN, DH), lambda i: (i, 0)),
    ],
    out_shape=[
        jax.ShapeDtypeStruct((N, D), jnp.float32),
        jax.ShapeDtypeStruct((N, DH), jnp.float32),
        jax.ShapeDtypeStruct((N, DH), jnp.float32),
    ],
)

_colss = pl.pallas_call(
    _colss_body,
    grid=(KHOPS + 1, _NB),
    in_specs=[pl.BlockSpec((1, BN, D), lambda j, i: (j, i, 0))],
    out_specs=pl.BlockSpec((1, 1, D), lambda j, i: (j, 0, 0)),
    out_shape=jax.ShapeDtypeStruct((KHOPS + 1, 1, D), jnp.float32),
)

_scale = pl.pallas_call(
    _scale_body,
    grid=(KHOPS + 1, _NB),
    in_specs=[
        pl.BlockSpec((1, BN, D), lambda j, i: (j, i, 0)),
        pl.BlockSpec((1, 1, D), lambda j, i: (j, 0, 0)),
    ],
    out_specs=pl.BlockSpec((1, BN, D), lambda j, i: (j, i, 0)),
    out_shape=jax.ShapeDtypeStruct((KHOPS + 1, N, D), jnp.float32),
)


def kernel(x, edge_index):
    row = edge_index[0].astype(jnp.int32)
    col = edge_index[1].astype(jnp.int32)
    def _pack(a, fill):
        a0 = a[:E0].reshape(NS, C0, CW)
        a0 = jnp.pad(a0, ((0, 0), (0, NCHUNKM - C0), (0, 0)),
                     constant_values=fill)
        cap1 = NS * C1 * CW
        a1 = jnp.concatenate(
            [a[E0:], jnp.full((cap1 - (E - E0),), fill, a.dtype)])
        a1 = a1.reshape(NS, C1, CW)
        a1 = jnp.pad(a1, ((0, 0), (0, NCHUNKM - C1), (0, 0)),
                     constant_values=fill)
        return jnp.concatenate([a0, a1], axis=0)

    rowdeg = _pack(row, N)
    srcr = _pack(row, 0)
    dstr = _pack(col, N)

    zerosd = jnp.zeros((N, DH), jnp.float32)
    onesd = jnp.ones((CW, DH), jnp.float32)

    degp = _deg_sc(rowdeg, zerosd, onesd)
    deg = degp[0, :, 0] + degp[1, :, 0]
    dinv = jnp.where(deg > 0, 1.0 / jnp.sqrt(deg), 0.0).reshape(N, 1)

    h, g0, g1 = _rownorm(x, dinv)
    lxs = [h]
    for _ in range(KHOPS):
        sp = _spmv_sc(g0, g1, srcr, dstr, zerosd)
        h, g0, g1 = _update(h, dinv, sp)
        lxs.append(h)

    lx = jnp.stack(lxs, axis=0)
    ss = _colss(lx)
    scl = 1.0 / jnp.maximum(jnp.sqrt(ss), EPS)
    return _scale(lx, scl)


# speedup vs baseline: 1.0613x; 1.0613x over previous
import functools

import jax
import jax.numpy as jnp
from jax import lax
from jax.experimental import pallas as pl
from jax.experimental.pallas import tpu as pltpu
from jax.experimental.pallas import tpu_sc as plsc

N = 10000
E = 320000
D = 128
KHOPS = 4
DH = D // 2

NC = 2
NS = 16
NW = NC * NS
CW = 128
NBUF = 2
F0 = 0.70
C0 = int(round(E * F0 / (NS * CW * NBUF))) * NBUF
E0 = NS * C0 * CW
C1 = (-(-(E - E0) // (NS * CW * NBUF))) * NBUF
NCHUNKM = max(C0, C1)
SLAB = 632
SLABL = N - (NS - 1) * SLAB
NT = N + 8

BN = 1000
EPS = 1e-12

_SC_PARAMS = pltpu.CompilerParams(use_tc_tiling_on_sc=False)


def _per_tile_slab(s, fn):
    off = pl.multiple_of(s * SLAB, 8)

    @pl.when(s < NS - 1)
    def _():
        fn(off, SLAB)

    @pl.when(s == NS - 1)
    def _():
        fn(off, SLABL)


def _sc_mesh():
    return plsc.VectorSubcoreMesh(
        core_axis_name="c", subcore_axis_name="s",
        num_cores=NC, num_subcores=NS)


DW = 16


@functools.cache
def _build_deg_sc():
    return pl.kernel(
        _deg_sc_body,
        out_type=jax.ShapeDtypeStruct((NC, N, DW), jnp.float32),
        mesh=_sc_mesh(),
        compiler_params=_SC_PARAMS,
        scratch_types=[
            pltpu.VMEM((NCHUNKM, CW), jnp.int32),
            pltpu.VMEM((CW, DW), jnp.float32),
            pltpu.VMEM_SHARED((NT, DW), jnp.float32),
        ],
    )


def _deg_sc_body(rowr, zerosd, onesd, out, idx_v, ones_v, deg_sh):
    c = lax.axis_index("c")
    s = lax.axis_index("s")
    t = c * NS + s
    pltpu.sync_copy(rowr.at[t], idx_v)
    pltpu.sync_copy(onesd, ones_v)
    _per_tile_slab(s, lambda off, sz: pltpu.sync_copy(
        zerosd.at[pl.ds(off, sz)], deg_sh.at[pl.ds(off, sz)]))
    plsc.subcore_barrier()

    def body(j, carry):
        pltpu.sync_copy(ones_v, deg_sh.at[idx_v.at[j]], add=True)
        return carry

    lax.fori_loop(0, lax.select(c == 0, C0, C1), body, 0)
    plsc.subcore_barrier()
    _per_tile_slab(s, lambda off, sz: pltpu.sync_copy(
        deg_sh.at[pl.ds(off, sz)], out.at[c, pl.ds(off, sz)]))
    return


@functools.cache
def _build_spmv_sc():
    return pl.kernel(
        _spmv_sc_body,
        out_type=jax.ShapeDtypeStruct((NC, 2, N, DH), jnp.float32),
        mesh=_sc_mesh(),
        compiler_params=_SC_PARAMS,
        scratch_types=[
            pltpu.VMEM((NCHUNKM, CW), jnp.int32),
            pltpu.VMEM((NCHUNKM, CW), jnp.int32),
            [pltpu.VMEM((CW, DH), jnp.float32)] * NBUF,
            pltpu.VMEM_SHARED((NT, DH), jnp.float32),
            [pltpu.SemaphoreType.DMA] * NBUF,
        ],
    )


def _spmv_sc_body(g0, g1, srcr, dstr, zerosd, out,
                  src_v, dst_v, bufs, s_sh, sems):
    c = lax.axis_index("c")
    s = lax.axis_index("s")
    t = c * NS + s
    pltpu.sync_copy(srcr.at[t], src_v)
    pltpu.sync_copy(dstr.at[t], dst_v)
    knt = lax.select(c == 0, C0 // NBUF, C1 // NBUF)
    for p, g in enumerate((g0, g1)):
        _per_tile_slab(s, lambda off, sz: pltpu.sync_copy(
            zerosd.at[pl.ds(off, sz)], s_sh.at[pl.ds(off, sz)]))
        plsc.subcore_barrier()
        for b in range(NBUF - 1):
            pltpu.async_copy(g.at[src_v.at[b]], bufs[b], sems[b])

        def body(k, carry):
            for b in range(NBUF):
                j = NBUF * k + b
                nb = (b + NBUF - 1) % NBUF
                pltpu.make_async_copy(g.at[src_v.at[j]], bufs[b],
                                      sems[b]).wait()
                if b == 0:
                    pltpu.async_copy(g.at[src_v.at[j + NBUF - 1]],
                                     bufs[nb], sems[nb])
                else:
                    @pl.when(k < knt - 1)
                    def _():
                        pltpu.async_copy(g.at[src_v.at[j + NBUF - 1]],
                                         bufs[nb], sems[nb])
                pltpu.sync_copy(bufs[b], s_sh.at[dst_v.at[j]], add=True)
            return carry

        lax.fori_loop(0, knt, body, 0)
        plsc.subcore_barrier()
        _per_tile_slab(s, lambda off, sz: pltpu.sync_copy(
            s_sh.at[pl.ds(off, sz)], out.at[c, p, pl.ds(off, sz)]))
        plsc.subcore_barrier()
    return


def _deg_sc(*args):
    return _build_deg_sc()(*args)


def _spmv_sc(*args):
    return _build_spmv_sc()(*args)


def _rownorm_body(x_ref, dinv_ref, h_ref, g0_ref, g1_ref):
    xb = x_ref[...]
    ss = jnp.sum(xb * xb, axis=1, keepdims=True)
    inv = 1.0 / jnp.maximum(jnp.sqrt(ss), EPS)
    hb = xb * inv
    h_ref[...] = hb
    gb = hb * dinv_ref[...]
    g0_ref[...] = gb[:, :DH]
    g1_ref[...] = gb[:, DH:]


def _update_body(h_ref, dinv_ref, sp_ref, hn_ref, g0_ref, g1_ref):
    hb = h_ref[...]
    dv = dinv_ref[...]
    s = sp_ref[0] + sp_ref[1]
    sfull = jnp.concatenate([s[0], s[1]], axis=1)
    hn = 2.0 * hb - dv * sfull
    hn_ref[...] = hn
    gb = hn * dv
    g0_ref[...] = gb[:, :DH]
    g1_ref[...] = gb[:, DH:]


def _colss_body(l0, l1, l2, l3, l4, ss_ref):
    i = pl.program_id(0)

    @pl.when(i == 0)
    def _():
        ss_ref[...] = jnp.zeros_like(ss_ref)

    for jj, lref in enumerate((l0, l1, l2, l3, l4)):
        xb = lref[...]
        ss_ref[jj, :, :] += jnp.sum(xb * xb, axis=0, keepdims=True)


def _scale_body(l0, l1, l2, l3, l4, scl_ref, o_ref):
    for jj, lref in enumerate((l0, l1, l2, l3, l4)):
        o_ref[jj, :, :] = lref[...] * scl_ref[jj]


_NB = N // BN

_rownorm = pl.pallas_call(
    _rownorm_body,
    grid=(_NB,),
    in_specs=[
        pl.BlockSpec((BN, D), lambda i: (i, 0)),
        pl.BlockSpec((BN, 1), lambda i: (i, 0)),
    ],
    out_specs=[
        pl.BlockSpec((BN, D), lambda i: (i, 0)),
        pl.BlockSpec((BN, DH), lambda i: (i, 0)),
        pl.BlockSpec((BN, DH), lambda i: (i, 0)),
    ],
    out_shape=[
        jax.ShapeDtypeStruct((N, D), jnp.float32),
        jax.ShapeDtypeStruct((N, DH), jnp.float32),
        jax.ShapeDtypeStruct((N, DH), jnp.float32),
    ],
)

_update = pl.pallas_call(
    _update_body,
    grid=(_NB,),
    in_specs=[
        pl.BlockSpec((BN, D), lambda i: (i, 0)),
        pl.BlockSpec((BN, 1), lambda i: (i, 0)),
        pl.BlockSpec((NC, 2, BN, DH), lambda i: (0, 0, i, 0)),
    ],
    out_specs=[
        pl.BlockSpec((BN, D), lambda i: (i, 0)),
        pl.BlockSpec((BN, DH), lambda i: (i, 0)),
        pl.BlockSpec((BN, DH), lambda i: (i, 0)),
    ],
    out_shape=[
        jax.ShapeDtypeStruct((N, D), jnp.float32),
        jax.ShapeDtypeStruct((N, DH), jnp.float32),
        jax.ShapeDtypeStruct((N, DH), jnp.float32),
    ],
)

_colss = pl.pallas_call(
    _colss_body,
    grid=(_NB,),
    in_specs=[pl.BlockSpec((BN, D), lambda i: (i, 0))] * (KHOPS + 1),
    out_specs=pl.BlockSpec((KHOPS + 1, 1, D), lambda i: (0, 0, 0)),
    out_shape=jax.ShapeDtypeStruct((KHOPS + 1, 1, D), jnp.float32),
)

_scale = pl.pallas_call(
    _scale_body,
    grid=(_NB,),
    in_specs=[pl.BlockSpec((BN, D), lambda i: (i, 0))] * (KHOPS + 1)
    + [pl.BlockSpec((KHOPS + 1, 1, D), lambda i: (0, 0, 0))],
    out_specs=pl.BlockSpec((KHOPS + 1, BN, D), lambda i: (0, i, 0)),
    out_shape=jax.ShapeDtypeStruct((KHOPS + 1, N, D), jnp.float32),
)


def kernel(x, edge_index):
    row = edge_index[0].astype(jnp.int32)
    col = edge_index[1].astype(jnp.int32)
    def _pack(a, fill):
        a0 = a[:E0].reshape(NS, C0, CW)
        a0 = jnp.pad(a0, ((0, 0), (0, NCHUNKM - C0), (0, 0)),
                     constant_values=fill)
        cap1 = NS * C1 * CW
        a1 = jnp.concatenate(
            [a[E0:], jnp.full((cap1 - (E - E0),), fill, a.dtype)])
        a1 = a1.reshape(NS, C1, CW)
        a1 = jnp.pad(a1, ((0, 0), (0, NCHUNKM - C1), (0, 0)),
                     constant_values=fill)
        return jnp.concatenate([a0, a1], axis=0)

    rowdeg = _pack(row, N)
    srcr = _pack(row, 0)
    dstr = _pack(col, N)

    zerosd = jnp.zeros((N, DH), jnp.float32)
    zerosw = jnp.zeros((N, DW), jnp.float32)
    onesw = jnp.ones((CW, DW), jnp.float32)

    degp = _deg_sc(rowdeg, zerosw, onesw)
    deg = degp[0, :, 0] + degp[1, :, 0]
    dinv = jnp.where(deg > 0, 1.0 / jnp.sqrt(deg), 0.0).reshape(N, 1)

    h, g0, g1 = _rownorm(x, dinv)
    lxs = [h]
    for _ in range(KHOPS):
        sp = _spmv_sc(g0, g1, srcr, dstr, zerosd)
        h, g0, g1 = _update(h, dinv, sp)
        lxs.append(h)

    ss = _colss(*lxs)
    scl = 1.0 / jnp.maximum(jnp.sqrt(ss), EPS)
    return _scale(*lxs, scl)


# speedup vs baseline: 1.0818x; 1.0193x over previous
import functools

import jax
import jax.numpy as jnp
from jax import lax
from jax.experimental import pallas as pl
from jax.experimental.pallas import tpu as pltpu
from jax.experimental.pallas import tpu_sc as plsc

N = 10000
E = 320000
D = 128
KHOPS = 4
DH = D // 2

NC = 2
NS = 16
NW = NC * NS
CW = 128
NBUF = 2
F0 = 0.72
C0 = int(round(E * F0 / (NS * CW * NBUF))) * NBUF
E0 = NS * C0 * CW
C1 = (-(-(E - E0) // (NS * CW * NBUF))) * NBUF
NCHUNKM = max(C0, C1)
SLAB = 632
SLABL = N - (NS - 1) * SLAB
NT = N + 8

BN = 1000
EPS = 1e-12

_SC_PARAMS = pltpu.CompilerParams(use_tc_tiling_on_sc=False)


def _per_tile_slab(s, fn):
    off = pl.multiple_of(s * SLAB, 8)

    @pl.when(s < NS - 1)
    def _():
        fn(off, SLAB)

    @pl.when(s == NS - 1)
    def _():
        fn(off, SLABL)


def _sc_mesh():
    return plsc.VectorSubcoreMesh(
        core_axis_name="c", subcore_axis_name="s",
        num_cores=NC, num_subcores=NS)


DW = 16


@functools.cache
def _build_deg_sc():
    return pl.kernel(
        _deg_sc_body,
        out_type=jax.ShapeDtypeStruct((NC, N, DW), jnp.float32),
        mesh=_sc_mesh(),
        compiler_params=_SC_PARAMS,
        scratch_types=[
            pltpu.VMEM((NCHUNKM, CW), jnp.int32),
            pltpu.VMEM((CW, DW), jnp.float32),
            pltpu.VMEM_SHARED((NT, DW), jnp.float32),
        ],
    )


def _deg_sc_body(rowr, zerosd, onesd, out, idx_v, ones_v, deg_sh):
    c = lax.axis_index("c")
    s = lax.axis_index("s")
    t = c * NS + s
    pltpu.sync_copy(rowr.at[t], idx_v)
    pltpu.sync_copy(onesd, ones_v)
    _per_tile_slab(s, lambda off, sz: pltpu.sync_copy(
        zerosd.at[pl.ds(off, sz)], deg_sh.at[pl.ds(off, sz)]))
    plsc.subcore_barrier()

    def body(j, carry):
        pltpu.sync_copy(ones_v, deg_sh.at[idx_v.at[j]], add=True)
        return carry

    lax.fori_loop(0, lax.select(c == 0, C0, C1), body, 0)
    plsc.subcore_barrier()
    _per_tile_slab(s, lambda off, sz: pltpu.sync_copy(
        deg_sh.at[pl.ds(off, sz)], out.at[c, pl.ds(off, sz)]))
    return


@functools.cache
def _build_spmv_sc():
    return pl.kernel(
        _spmv_sc_body,
        out_type=jax.ShapeDtypeStruct((NC, 2, N, DH), jnp.float32),
        mesh=_sc_mesh(),
        compiler_params=_SC_PARAMS,
        scratch_types=[
            pltpu.VMEM((NCHUNKM, CW), jnp.int32),
            pltpu.VMEM((NCHUNKM, CW), jnp.int32),
            [pltpu.VMEM((CW, DH), jnp.float32)] * NBUF,
            pltpu.VMEM_SHARED((NT, DH), jnp.float32),
            [pltpu.SemaphoreType.DMA] * NBUF,
        ],
    )


def _spmv_sc_body(g0, g1, srcr, dstr, zerosd, out,
                  src_v, dst_v, bufs, s_sh, sems):
    c = lax.axis_index("c")
    s = lax.axis_index("s")
    t = c * NS + s
    pltpu.sync_copy(srcr.at[t], src_v)
    pltpu.sync_copy(dstr.at[t], dst_v)
    knt = lax.select(c == 0, C0 // NBUF, C1 // NBUF)
    for p, g in enumerate((g0, g1)):
        _per_tile_slab(s, lambda off, sz: pltpu.sync_copy(
            zerosd.at[pl.ds(off, sz)], s_sh.at[pl.ds(off, sz)]))
        plsc.subcore_barrier()
        for b in range(NBUF - 1):
            pltpu.async_copy(g.at[src_v.at[b]], bufs[b], sems[b])

        def body(k, carry):
            for b in range(NBUF):
                j = NBUF * k + b
                nb = (b + NBUF - 1) % NBUF
                pltpu.make_async_copy(g.at[src_v.at[j]], bufs[b],
                                      sems[b]).wait()
                if b == 0:
                    pltpu.async_copy(g.at[src_v.at[j + NBUF - 1]],
                                     bufs[nb], sems[nb])
                else:
                    @pl.when(k < knt - 1)
                    def _():
                        pltpu.async_copy(g.at[src_v.at[j + NBUF - 1]],
                                         bufs[nb], sems[nb])
                pltpu.sync_copy(bufs[b], s_sh.at[dst_v.at[j]], add=True)
            return carry

        lax.fori_loop(0, knt, body, 0)
        plsc.subcore_barrier()
        _per_tile_slab(s, lambda off, sz: pltpu.sync_copy(
            s_sh.at[pl.ds(off, sz)], out.at[c, p, pl.ds(off, sz)]))
        plsc.subcore_barrier()
    return


def _deg_sc(*args):
    return _build_deg_sc()(*args)


def _spmv_sc(*args):
    return _build_spmv_sc()(*args)


def _rownorm_body(x_ref, dinv_ref, h_ref, g0_ref, g1_ref):
    xb = x_ref[...]
    ss = jnp.sum(xb * xb, axis=1, keepdims=True)
    inv = 1.0 / jnp.maximum(jnp.sqrt(ss), EPS)
    hb = xb * inv
    h_ref[...] = hb
    gb = hb * dinv_ref[...]
    g0_ref[...] = gb[:, :DH]
    g1_ref[...] = gb[:, DH:]


def _update_body(h_ref, dinv_ref, sp_ref, hn_ref, g0_ref, g1_ref):
    hb = h_ref[...]
    dv = dinv_ref[...]
    s = sp_ref[0] + sp_ref[1]
    sfull = jnp.concatenate([s[0], s[1]], axis=1)
    hn = 2.0 * hb - dv * sfull
    hn_ref[...] = hn
    gb = hn * dv
    g0_ref[...] = gb[:, :DH]
    g1_ref[...] = gb[:, DH:]


def _colss_body(l0, l1, l2, l3, l4, ss_ref):
    i = pl.program_id(0)

    @pl.when(i == 0)
    def _():
        ss_ref[...] = jnp.zeros_like(ss_ref)

    for jj, lref in enumerate((l0, l1, l2, l3, l4)):
        xb = lref[...]
        ss_ref[jj, :, :] += jnp.sum(xb * xb, axis=0, keepdims=True)


def _scale_body(l0, l1, l2, l3, l4, scl_ref, o_ref):
    for jj, lref in enumerate((l0, l1, l2, l3, l4)):
        o_ref[jj, :, :] = lref[...] * scl_ref[jj]


_NB = N // BN

_rownorm = pl.pallas_call(
    _rownorm_body,
    grid=(_NB,),
    in_specs=[
        pl.BlockSpec((BN, D), lambda i: (i, 0)),
        pl.BlockSpec((BN, 1), lambda i: (i, 0)),
    ],
    out_specs=[
        pl.BlockSpec((BN, D), lambda i: (i, 0)),
        pl.BlockSpec((BN, DH), lambda i: (i, 0)),
        pl.BlockSpec((BN, DH), lambda i: (i, 0)),
    ],
    out_shape=[
        jax.ShapeDtypeStruct((N, D), jnp.float32),
        jax.ShapeDtypeStruct((N, DH), jnp.float32),
        jax.ShapeDtypeStruct((N, DH), jnp.float32),
    ],
)

_update = pl.pallas_call(
    _update_body,
    grid=(_NB,),
    in_specs=[
        pl.BlockSpec((BN, D), lambda i: (i, 0)),
        pl.BlockSpec((BN, 1), lambda i: (i, 0)),
        pl.BlockSpec((NC, 2, BN, DH), lambda i: (0, 0, i, 0)),
    ],
    out_specs=[
        pl.BlockSpec((BN, D), lambda i: (i, 0)),
        pl.BlockSpec((BN, DH), lambda i: (i, 0)),
        pl.BlockSpec((BN, DH), lambda i: (i, 0)),
    ],
    out_shape=[
        jax.ShapeDtypeStruct((N, D), jnp.float32),
        jax.ShapeDtypeStruct((N, DH), jnp.float32),
        jax.ShapeDtypeStruct((N, DH), jnp.float32),
    ],
)

_colss = pl.pallas_call(
    _colss_body,
    grid=(_NB,),
    in_specs=[pl.BlockSpec((BN, D), lambda i: (i, 0))] * (KHOPS + 1),
    out_specs=pl.BlockSpec((KHOPS + 1, 1, D), lambda i: (0, 0, 0)),
    out_shape=jax.ShapeDtypeStruct((KHOPS + 1, 1, D), jnp.float32),
)

_scale = pl.pallas_call(
    _scale_body,
    grid=(_NB,),
    in_specs=[pl.BlockSpec((BN, D), lambda i: (i, 0))] * (KHOPS + 1)
    + [pl.BlockSpec((KHOPS + 1, 1, D), lambda i: (0, 0, 0))],
    out_specs=pl.BlockSpec((KHOPS + 1, BN, D), lambda i: (0, i, 0)),
    out_shape=jax.ShapeDtypeStruct((KHOPS + 1, N, D), jnp.float32),
)


def kernel(x, edge_index):
    row = edge_index[0].astype(jnp.int32)
    col = edge_index[1].astype(jnp.int32)
    def _pack(a, fill):
        a0 = a[:E0].reshape(NS, C0, CW)
        a0 = jnp.pad(a0, ((0, 0), (0, NCHUNKM - C0), (0, 0)),
                     constant_values=fill)
        cap1 = NS * C1 * CW
        a1 = jnp.concatenate(
            [a[E0:], jnp.full((cap1 - (E - E0),), fill, a.dtype)])
        a1 = a1.reshape(NS, C1, CW)
        a1 = jnp.pad(a1, ((0, 0), (0, NCHUNKM - C1), (0, 0)),
                     constant_values=fill)
        return jnp.concatenate([a0, a1], axis=0)

    rowdeg = _pack(row, N)
    srcr = _pack(row, 0)
    dstr = _pack(col, N)

    zerosd = jnp.zeros((N, DH), jnp.float32)
    zerosw = jnp.zeros((N, DW), jnp.float32)
    onesw = jnp.ones((CW, DW), jnp.float32)

    degp = _deg_sc(rowdeg, zerosw, onesw)
    deg = degp[0, :, 0] + degp[1, :, 0]
    dinv = jnp.where(deg > 0, 1.0 / jnp.sqrt(deg), 0.0).reshape(N, 1)

    h, g0, g1 = _rownorm(x, dinv)
    lxs = [h]
    for _ in range(KHOPS):
        sp = _spmv_sc(g0, g1, srcr, dstr, zerosd)
        h, g0, g1 = _update(h, dinv, sp)
        lxs.append(h)

    ss = _colss(*lxs)
    scl = 1.0 / jnp.maximum(jnp.sqrt(ss), EPS)
    return _scale(*lxs, scl)
